# baseline (speedup 1.0000x reference)
import functools

import jax
import jax.numpy as jnp
from jax import lax
from jax.experimental import pallas as pl
from jax.experimental.pallas import tpu as pltpu
from jax.experimental.pallas import tpu_sc as plsc

N, D, H, DH = 2048, 768, 12, 64
E, HID = 8, 3072
EP = 128
RB = 512
NEG = -1e30

GB = 256
NBLK = (2 * N) // GB + E - 1
P = NBLK * GB
A = 2 * N
CH = 512

NC, NS = 2, 16
NW = NC * NS
TPW = N // NW


def _mmb(a, b):
    return lax.dot_general(
        a.astype(jnp.bfloat16), b.astype(jnp.bfloat16),
        (((1,), (0,)), ((), ())), preferred_element_type=jnp.float32)

def _ln(x, g, b):
    m = jnp.mean(x, axis=-1, keepdims=True)
    v = jnp.mean((x - m) ** 2, axis=-1, keepdims=True)
    return (x - m) / jnp.sqrt(v + 1e-5) * g + b


def _ln_qkv_body(x_ref, g_ref, b_ref, w_ref, bias_ref, o_ref):
    xn = _ln(x_ref[...], g_ref[...], b_ref[...])
    o_ref[...] = (
        jnp.dot(xn, w_ref[...], preferred_element_type=jnp.float32) + bias_ref[...]
    )


def _attn_body(q_ref, k_ref, v_ref, o_ref):
    q = q_ref[0]
    k = k_ref[0]
    v = v_ref[0]
    s = lax.dot_general(
        q, k, (((1,), (1,)), ((), ())), preferred_element_type=jnp.float32
    ) * (DH ** -0.5)
    m = jnp.max(s, axis=-1, keepdims=True)
    p = jnp.exp(s - m)
    p = p / jnp.sum(p, axis=-1, keepdims=True)
    o_ref[0] = jnp.dot(p, v, preferred_element_type=jnp.float32)


def _post_body(a_ref, wp_ref, bp_ref, x_ref, g_ref, b_ref, wr_ref,
               x2_ref, ridx_ref, rwt_ref):
    t = (
        jnp.dot(a_ref[...], wp_ref[...], preferred_element_type=jnp.float32)
        + bp_ref[...]
        + x_ref[...]
    )
    x2 = _ln(t, g_ref[...], b_ref[...])
    x2_ref[...] = x2
    logits = jnp.dot(x2, wr_ref[...], preferred_element_type=jnp.float32)
    lane = lax.broadcasted_iota(jnp.int32, logits.shape, 1)
    logits = jnp.where(lane < E, logits, NEG)
    big = jnp.int32(10**9)
    m1 = jnp.max(logits, axis=-1, keepdims=True)
    i1 = jnp.min(jnp.where(logits == m1, lane, big), axis=-1, keepdims=True)
    l2 = jnp.where(lane == i1, NEG, logits)
    m2 = jnp.max(l2, axis=-1, keepdims=True)
    i2 = jnp.min(jnp.where(l2 == m2, lane, big), axis=-1, keepdims=True)
    e2 = jnp.exp(m2 - m1)
    w1 = 2.0 / (1.0 + e2)
    w2 = 2.0 * e2 / (1.0 + e2)
    ridx_ref[...] = jnp.where(lane == 0, i1, jnp.where(lane == 1, i2, 0))
    rwt_ref[...] = jnp.where(lane == 0, w1, jnp.where(lane == 1, w2, 0.0))


def _meta_body(ridx_ref, rwt_ref, pos_ref, meta_ref, w0_ref, w1_ref):
    ridx = ridx_ref[...]
    rwt = rwt_ref[...]
    i1 = ridx[:, 0:1]
    i2 = ridx[:, 1:2]
    e_col = jnp.concatenate([i1, i2], axis=0)
    lane = lax.broadcasted_iota(jnp.int32, (A, EP), 1)
    oh = (lane == e_col).astype(jnp.float32)

    r = lax.broadcasted_iota(jnp.int32, (CH, CH), 0)
    c = lax.broadcasted_iota(jnp.int32, (CH, CH), 1)
    L = (r >= c).astype(jnp.float32)
    carry = jnp.zeros((1, EP), jnp.float32)
    parts = []
    for k in range(A // CH):
        ohc = oh[k * CH:(k + 1) * CH]
        inc = jnp.dot(L, ohc, preferred_element_type=jnp.float32) + carry
        parts.append(inc)
        carry = inc[CH - 1:CH, :]
    incl = jnp.concatenate(parts, axis=0)
    cnt = carry
    padded = jnp.floor((cnt + (GB - 1)) / GB) * GB
    r8 = lax.broadcasted_iota(jnp.int32, (EP, EP), 0)
    c8 = lax.broadcasted_iota(jnp.int32, (EP, EP), 1)
    U = (r8 < c8).astype(jnp.float32)
    base = jnp.dot(padded, U, preferred_element_type=jnp.float32)

    sel = lane == e_col
    pos = jnp.sum(jnp.where(sel, incl - 1.0 + base, 0.0), axis=1, keepdims=True)
    pos_ref[...] = jnp.broadcast_to(pos, (A, EP)).astype(jnp.int32)

    ident = (r8 == c8).astype(jnp.float32)
    base_col = lax.dot_general(
        ident, base, (((1,), (1,)), ((), ())), preferred_element_type=jnp.float32
    )
    padded_col = lax.dot_general(
        ident, padded, (((1,), (1,)), ((), ())), preferred_element_type=jnp.float32
    )
    bstart = (c8 * GB).astype(jnp.float32)
    rowe = r8.astype(jnp.float32)
    ind = (base_col <= bstart) & (bstart < base_col + padded_col) & (r8 < E)
    own = jnp.sum(jnp.where(ind, rowe, 0.0), axis=0, keepdims=True)
    act = jnp.sum(ind.astype(jnp.float32), axis=0, keepdims=True)
    tot_b = jnp.sum(padded, axis=1, keepdims=True) / GB
    lanei = lax.broadcasted_iota(jnp.int32, (1, EP), 1).astype(jnp.float32)
    last_own = jnp.sum(
        jnp.where(lanei == tot_b - 1.0, own, 0.0), axis=1, keepdims=True
    )
    clamped = jnp.where(act > 0, own, last_own)
    row8 = lax.broadcasted_iota(jnp.int32, (8, EP), 0)
    meta = jnp.where(row8 == 0, clamped, jnp.where(row8 == 1, act, 0.0))
    meta_ref[...] = meta.astype(jnp.int32)

    w0_ref[...] = jnp.broadcast_to(rwt[:, 0:1], (N, EP))
    w1_ref[...] = jnp.broadcast_to(rwt[:, 1:2], (N, EP))


def _sca_body(x2_hbm, pos0_hbm, pos1_hbm, w0_hbm, w1_hbm,
              xg_hbm, wg_hbm, idx0, idx1, rows, wr0, wr1, sem):
    wid = lax.axis_index("s") * NC + lax.axis_index("c")
    base = wid * TPW
    pltpu.sync_copy(pos0_hbm.at[pl.ds(base, TPW)], idx0)
    pltpu.sync_copy(pos1_hbm.at[pl.ds(base, TPW)], idx1)
    pltpu.sync_copy(x2_hbm.at[pl.ds(base, TPW)], rows)
    pltpu.sync_copy(w0_hbm.at[pl.ds(base, TPW)], wr0)
    pltpu.sync_copy(w1_hbm.at[pl.ds(base, TPW)], wr1)
    pltpu.async_copy(rows, xg_hbm.at[idx0], sem).wait()
    pltpu.async_copy(rows, xg_hbm.at[idx1], sem).wait()
    pltpu.async_copy(wr0, wg_hbm.at[idx0], sem).wait()
    pltpu.async_copy(wr1, wg_hbm.at[idx1], sem).wait()


def _sc_scatter(x2, pos0, pos1, w0rep, w1rep):
    mesh = plsc.VectorSubcoreMesh(
        core_axis_name="c", subcore_axis_name="s", num_cores=NC, num_subcores=NS
    )
    f = functools.partial(
        pl.kernel,
        out_type=[
            jax.ShapeDtypeStruct((P, D), jnp.float32),
            jax.ShapeDtypeStruct((P, EP), jnp.float32),
        ],
        mesh=mesh,
        scratch_types=[
            pltpu.VMEM((TPW,), jnp.int32),
            pltpu.VMEM((TPW,), jnp.int32),
            pltpu.VMEM((TPW, D), jnp.float32),
            pltpu.VMEM((TPW, EP), jnp.float32),
            pltpu.VMEM((TPW, EP), jnp.float32),
            pltpu.SemaphoreType.DMA,
        ],
    )(_sca_body)
    return f(x2, pos0, pos1, w0rep, w1rep)


def _ffn_body(m_ref, xg_ref, wg_ref, w1_ref, b1_ref, w2_ref, b2_ref, yg_ref):
    b = pl.program_id(0)
    c = pl.program_id(1)

    @pl.when(m_ref[b, 1] == 1)
    def _():
        h = _mmb(xg_ref[...], w1_ref[0]) + b1_ref[0, 0]
        h = jax.nn.gelu(h)
        y = _mmb(h, w2_ref[0])
        w_col = wg_ref[:, 0:1]

        @pl.when(c == 0)
        def _():
            yg_ref[...] = (y + b2_ref[0, 0]) * w_col

        @pl.when(c == 1)
        def _():
            yg_ref[...] += y * w_col


def _scb_body(yg_hbm, pos0_hbm, pos1_hbm, g0_hbm, g1_hbm, idx0, idx1, r0, r1, sem):
    wid = lax.axis_index("s") * NC + lax.axis_index("c")
    base = wid * TPW
    pltpu.sync_copy(pos0_hbm.at[pl.ds(base, TPW)], idx0)
    pltpu.sync_copy(pos1_hbm.at[pl.ds(base, TPW)], idx1)
    pltpu.async_copy(yg_hbm.at[idx0], r0, sem).wait()
    pltpu.async_copy(yg_hbm.at[idx1], r1, sem).wait()
    pltpu.sync_copy(r0, g0_hbm.at[pl.ds(base, TPW)])
    pltpu.sync_copy(r1, g1_hbm.at[pl.ds(base, TPW)])


def _sc_gather(yg, pos0, pos1):
    mesh = plsc.VectorSubcoreMesh(
        core_axis_name="c", subcore_axis_name="s", num_cores=NC, num_subcores=NS
    )
    f = functools.partial(
        pl.kernel,
        out_type=[
            jax.ShapeDtypeStruct((N, D), jnp.float32),
            jax.ShapeDtypeStruct((N, D), jnp.float32),
        ],
        mesh=mesh,
        scratch_types=[
            pltpu.VMEM((TPW,), jnp.int32),
            pltpu.VMEM((TPW,), jnp.int32),
            pltpu.VMEM((TPW, D), jnp.float32),
            pltpu.VMEM((TPW, D), jnp.float32),
            pltpu.SemaphoreType.DMA,
        ],
    )(_scb_body)
    return f(yg, pos0, pos1)


def _add_body(a_ref, b_ref, o_ref):
    o_ref[...] = a_ref[...] + b_ref[...]


def kernel(x, g1, bn1, Wqkv, bqkv, Wp, bp, g2, bn2, Wr, W1, b1, W2, b2):
    xf = x.reshape(N, D)
    g1r, bn1r = g1.reshape(1, D), bn1.reshape(1, D)
    g2r, bn2r = g2.reshape(1, D), bn2.reshape(1, D)
    bqkvr = bqkv.reshape(1, 3 * D)
    bpr = bp.reshape(1, D)
    Wr_pad = jnp.pad(Wr, ((0, 0), (0, EP - E)))
    b1r = b1.reshape(E, 2, 1, HID // 2)
    b2r = b2.reshape(E, 1, 1, D)

    qkv = pl.pallas_call(
        _ln_qkv_body,
        grid=(N // RB, 3),
        in_specs=[
            pl.BlockSpec((RB, D), lambda i, c: (i, 0)),
            pl.BlockSpec((1, D), lambda i, c: (0, 0)),
            pl.BlockSpec((1, D), lambda i, c: (0, 0)),
            pl.BlockSpec((D, D), lambda i, c: (0, c)),
            pl.BlockSpec((1, D), lambda i, c: (0, c)),
        ],
        out_specs=pl.BlockSpec((RB, D), lambda i, c: (i, c)),
        out_shape=jax.ShapeDtypeStruct((N, 3 * D), jnp.float32),
    )(xf, g1r, bn1r, Wqkv, bqkvr)

    q3 = qkv[:, :D].reshape(N, H, DH).transpose(1, 0, 2)
    k3 = qkv[:, D:2 * D].reshape(N, H, DH).transpose(1, 0, 2)
    v3 = qkv[:, 2 * D:].reshape(N, H, DH).transpose(1, 0, 2)

    o3 = pl.pallas_call(
        _attn_body,
        grid=(H, N // RB),
        in_specs=[
            pl.BlockSpec((1, RB, DH), lambda h, i: (h, i, 0)),
            pl.BlockSpec((1, N, DH), lambda h, i: (h, 0, 0)),
            pl.BlockSpec((1, N, DH), lambda h, i: (h, 0, 0)),
        ],
        out_specs=pl.BlockSpec((1, RB, DH), lambda h, i: (h, i, 0)),
        out_shape=jax.ShapeDtypeStruct((H, N, DH), jnp.float32),
    )(q3, k3, v3)
    attno = o3.transpose(1, 0, 2).reshape(N, D)

    x2, ridx, rwt = pl.pallas_call(
        _post_body,
        grid=(N // RB,),
        in_specs=[
            pl.BlockSpec((RB, D), lambda i: (i, 0)),
            pl.BlockSpec((D, D), lambda i: (0, 0)),
            pl.BlockSpec((1, D), lambda i: (0, 0)),
            pl.BlockSpec((RB, D), lambda i: (i, 0)),
            pl.BlockSpec((1, D), lambda i: (0, 0)),
            pl.BlockSpec((1, D), lambda i: (0, 0)),
            pl.BlockSpec((D, EP), lambda i: (0, 0)),
        ],
        out_specs=[
            pl.BlockSpec((RB, D), lambda i: (i, 0)),
            pl.BlockSpec((RB, EP), lambda i: (i, 0)),
            pl.BlockSpec((RB, EP), lambda i: (i, 0)),
        ],
        out_shape=[
            jax.ShapeDtypeStruct((N, D), jnp.float32),
            jax.ShapeDtypeStruct((N, EP), jnp.int32),
            jax.ShapeDtypeStruct((N, EP), jnp.float32),
        ],
    )(attno, Wp, bpr, xf, g2r, bn2r, Wr_pad)

    pos, meta8, w0rep, w1rep = pl.pallas_call(
        _meta_body,
        grid=(1,),
        in_specs=[
            pl.BlockSpec((N, EP), lambda i: (0, 0)),
            pl.BlockSpec((N, EP), lambda i: (0, 0)),
        ],
        out_specs=[
            pl.BlockSpec((A, EP), lambda i: (0, 0)),
            pl.BlockSpec((8, EP), lambda i: (0, 0)),
            pl.BlockSpec((N, EP), lambda i: (0, 0)),
            pl.BlockSpec((N, EP), lambda i: (0, 0)),
        ],
        out_shape=[
            jax.ShapeDtypeStruct((A, EP), jnp.int32),
            jax.ShapeDtypeStruct((8, EP), jnp.int32),
            jax.ShapeDtypeStruct((N, EP), jnp.float32),
            jax.ShapeDtypeStruct((N, EP), jnp.float32),
        ],
    )(ridx, rwt)

    pos0 = pos[:N, 0]
    pos1 = pos[N:, 0]
    bmeta = meta8[:2, :NBLK].transpose(1, 0)

    xg, wg = _sc_scatter(x2, pos0, pos1, w0rep, w1rep)

    grid_spec = pltpu.PrefetchScalarGridSpec(
        num_scalar_prefetch=1,
        grid=(NBLK, 2),
        in_specs=[
            pl.BlockSpec((GB, D), lambda b, c, m: (b, 0)),
            pl.BlockSpec((GB, EP), lambda b, c, m: (b, 0)),
            pl.BlockSpec((1, D, HID // 2), lambda b, c, m: (m[b, 0], 0, c)),
            pl.BlockSpec((1, 1, 1, HID // 2), lambda b, c, m: (m[b, 0], c, 0, 0)),
            pl.BlockSpec((1, HID // 2, D), lambda b, c, m: (m[b, 0], c, 0)),
            pl.BlockSpec((1, 1, 1, D), lambda b, c, m: (m[b, 0], 0, 0, 0)),
        ],
        out_specs=pl.BlockSpec((GB, D), lambda b, c, m: (b, 0)),
    )
    yg = pl.pallas_call(
        _ffn_body,
        grid_spec=grid_spec,
        out_shape=jax.ShapeDtypeStruct((P, D), jnp.float32),
    )(bmeta, xg, wg, W1, b1r, W2, b2r)

    g0, g1 = _sc_gather(yg, pos0, pos1)

    out = pl.pallas_call(
        _add_body,
        grid=(N // RB,),
        in_specs=[
            pl.BlockSpec((RB, D), lambda i: (i, 0)),
            pl.BlockSpec((RB, D), lambda i: (i, 0)),
        ],
        out_specs=pl.BlockSpec((RB, D), lambda i: (i, 0)),
        out_shape=jax.ShapeDtypeStruct((N, D), jnp.float32),
    )(g0, g1)

    return out.reshape(1, N, D)


# speedup vs baseline: 1.2503x; 1.2503x over previous
import functools

import jax
import jax.numpy as jnp
from jax import lax
from jax.experimental import pallas as pl
from jax.experimental.pallas import tpu as pltpu
from jax.experimental.pallas import tpu_sc as plsc

N, D, H, DH = 2048, 768, 12, 64
E, HID = 8, 3072
EP = 128
RB = 512
NEG = -1e30

GB = 256
NBLK = (2 * N) // GB + E - 1
P = NBLK * GB
A = 2 * N
CH = 512

NC, NS = 2, 16
NW = NC * NS
TPW = N // NW


def _mmb(a, b):
    return lax.dot_general(
        a.astype(jnp.bfloat16), b.astype(jnp.bfloat16),
        (((1,), (0,)), ((), ())), preferred_element_type=jnp.float32)

def _mmb_nt(a, b):
    return lax.dot_general(
        a.astype(jnp.bfloat16), b.astype(jnp.bfloat16),
        (((1,), (1,)), ((), ())), preferred_element_type=jnp.float32)


def _ln(x, g, b):
    m = jnp.mean(x, axis=-1, keepdims=True)
    v = jnp.mean((x - m) ** 2, axis=-1, keepdims=True)
    return (x - m) / jnp.sqrt(v + 1e-5) * g + b


def _ln_qkv_body(x_ref, g_ref, b_ref, w_ref, bias_ref, o_ref):
    xn = _ln(x_ref[...], g_ref[...], b_ref[...])
    o_ref[...] = _mmb(xn, w_ref[...]) + bias_ref[...]


def _attn_body(q_ref, k_ref, v_ref, o_ref):
    outs = []
    for hh in range(2):
        sl = slice(hh * DH, (hh + 1) * DH)
        q = q_ref[:, sl] * (DH ** -0.5)
        k = k_ref[:, sl]
        v = v_ref[:, sl]
        s = _mmb_nt(q, k)
        m = jnp.max(s, axis=-1, keepdims=True)
        p = jnp.exp(s - m)
        p = p / jnp.sum(p, axis=-1, keepdims=True)
        outs.append(_mmb(p, v))
    o_ref[...] = jnp.concatenate(outs, axis=1)


def _post_body(a_ref, wp_ref, bp_ref, x_ref, g_ref, b_ref, wr_ref,
               x2_ref, ridx_ref, rwt_ref):
    t = x_ref[...] + (_mmb(a_ref[...], wp_ref[...]) + bp_ref[...])
    x2 = _ln(t, g_ref[...], b_ref[...])
    x2_ref[...] = x2
    logits = _mmb(x2, wr_ref[...])
    lane = lax.broadcasted_iota(jnp.int32, logits.shape, 1)
    logits = jnp.where(lane < E, logits, NEG)
    big = jnp.int32(10**9)
    m1 = jnp.max(logits, axis=-1, keepdims=True)
    i1 = jnp.min(jnp.where(logits == m1, lane, big), axis=-1, keepdims=True)
    l2 = jnp.where(lane == i1, NEG, logits)
    m2 = jnp.max(l2, axis=-1, keepdims=True)
    i2 = jnp.min(jnp.where(l2 == m2, lane, big), axis=-1, keepdims=True)
    e2 = jnp.exp(m2 - m1)
    w1 = 2.0 / (1.0 + e2)
    w2 = 2.0 * e2 / (1.0 + e2)
    ridx_ref[...] = jnp.where(lane == 0, i1, jnp.where(lane == 1, i2, 0))
    rwt_ref[...] = jnp.where(lane == 0, w1, jnp.where(lane == 1, w2, 0.0))


def _meta_body(ridx_ref, rwt_ref, pos_ref, meta_ref, w0_ref, w1_ref):
    ridx = ridx_ref[...]
    rwt = rwt_ref[...]
    i1 = ridx[:, 0:1]
    i2 = ridx[:, 1:2]
    e_col = jnp.concatenate([i1, i2], axis=0)
    lane = lax.broadcasted_iota(jnp.int32, (A, EP), 1)
    oh = (lane == e_col).astype(jnp.float32)

    r = lax.broadcasted_iota(jnp.int32, (CH, CH), 0)
    c = lax.broadcasted_iota(jnp.int32, (CH, CH), 1)
    L = (r >= c).astype(jnp.float32)
    carry = jnp.zeros((1, EP), jnp.float32)
    parts = []
    for k in range(A // CH):
        ohc = oh[k * CH:(k + 1) * CH]
        inc = jnp.dot(L, ohc, preferred_element_type=jnp.float32) + carry
        parts.append(inc)
        carry = inc[CH - 1:CH, :]
    incl = jnp.concatenate(parts, axis=0)
    cnt = carry
    padded = jnp.floor((cnt + (GB - 1)) / GB) * GB
    r8 = lax.broadcasted_iota(jnp.int32, (EP, EP), 0)
    c8 = lax.broadcasted_iota(jnp.int32, (EP, EP), 1)
    U = (r8 < c8).astype(jnp.float32)
    base = jnp.dot(padded, U, preferred_element_type=jnp.float32)

    sel = lane == e_col
    pos = jnp.sum(jnp.where(sel, incl - 1.0 + base, 0.0), axis=1, keepdims=True)
    pos_ref[...] = jnp.broadcast_to(pos, (A, EP)).astype(jnp.int32)

    ident = (r8 == c8).astype(jnp.float32)
    base_col = lax.dot_general(
        ident, base, (((1,), (1,)), ((), ())), preferred_element_type=jnp.float32
    )
    padded_col = lax.dot_general(
        ident, padded, (((1,), (1,)), ((), ())), preferred_element_type=jnp.float32
    )
    bstart = (c8 * GB).astype(jnp.float32)
    rowe = r8.astype(jnp.float32)
    ind = (base_col <= bstart) & (bstart < base_col + padded_col) & (r8 < E)
    own = jnp.sum(jnp.where(ind, rowe, 0.0), axis=0, keepdims=True)
    act = jnp.sum(ind.astype(jnp.float32), axis=0, keepdims=True)
    tot_b = jnp.sum(padded, axis=1, keepdims=True) / GB
    lanei = lax.broadcasted_iota(jnp.int32, (1, EP), 1).astype(jnp.float32)
    last_own = jnp.sum(
        jnp.where(lanei == tot_b - 1.0, own, 0.0), axis=1, keepdims=True
    )
    clamped = jnp.where(act > 0, own, last_own)
    row8 = lax.broadcasted_iota(jnp.int32, (8, EP), 0)
    meta = jnp.where(row8 == 0, clamped, jnp.where(row8 == 1, act, 0.0))
    meta_ref[...] = meta.astype(jnp.int32)

    w0_ref[...] = jnp.broadcast_to(rwt[:, 0:1], (N, EP))
    w1_ref[...] = jnp.broadcast_to(rwt[:, 1:2], (N, EP))


def _sca_body(x2_hbm, pos0_hbm, pos1_hbm, w0_hbm, w1_hbm,
              xg_hbm, wg_hbm, idx0, idx1, rows, wr0, wr1, sem):
    wid = lax.axis_index("s") * NC + lax.axis_index("c")
    base = wid * TPW
    pltpu.sync_copy(pos0_hbm.at[pl.ds(base, TPW)], idx0)
    pltpu.sync_copy(pos1_hbm.at[pl.ds(base, TPW)], idx1)
    pltpu.sync_copy(x2_hbm.at[pl.ds(base, TPW)], rows)
    pltpu.sync_copy(w0_hbm.at[pl.ds(base, TPW)], wr0)
    pltpu.sync_copy(w1_hbm.at[pl.ds(base, TPW)], wr1)
    pltpu.async_copy(rows, xg_hbm.at[idx0], sem).wait()
    pltpu.async_copy(rows, xg_hbm.at[idx1], sem).wait()
    pltpu.async_copy(wr0, wg_hbm.at[idx0], sem).wait()
    pltpu.async_copy(wr1, wg_hbm.at[idx1], sem).wait()


def _sc_scatter(x2, pos0, pos1, w0rep, w1rep):
    mesh = plsc.VectorSubcoreMesh(
        core_axis_name="c", subcore_axis_name="s", num_cores=NC, num_subcores=NS
    )
    f = functools.partial(
        pl.kernel,
        out_type=[
            jax.ShapeDtypeStruct((P, D), jnp.float32),
            jax.ShapeDtypeStruct((P, EP), jnp.float32),
        ],
        mesh=mesh,
        scratch_types=[
            pltpu.VMEM((TPW,), jnp.int32),
            pltpu.VMEM((TPW,), jnp.int32),
            pltpu.VMEM((TPW, D), jnp.float32),
            pltpu.VMEM((TPW, EP), jnp.float32),
            pltpu.VMEM((TPW, EP), jnp.float32),
            pltpu.SemaphoreType.DMA,
        ],
    )(_sca_body)
    return f(x2, pos0, pos1, w0rep, w1rep)


def _ffn_body(m_ref, xg_ref, wg_ref, w1_ref, b1_ref, w2_ref, b2_ref, yg_ref):
    b = pl.program_id(0)
    c = pl.program_id(1)

    @pl.when(m_ref[b, 1] == 1)
    def _():
        h = _mmb(xg_ref[...], w1_ref[0]) + b1_ref[0, 0]
        h = jax.nn.gelu(h)
        y = _mmb(h, w2_ref[0])
        w_col = wg_ref[:, 0:1]

        @pl.when(c == 0)
        def _():
            yg_ref[...] = (y + b2_ref[0, 0]) * w_col

        @pl.when(c == 1)
        def _():
            yg_ref[...] += y * w_col


def _scb_body(yg_hbm, pos0_hbm, pos1_hbm, g0_hbm, g1_hbm, idx0, idx1, r0, r1, sem):
    wid = lax.axis_index("s") * NC + lax.axis_index("c")
    base = wid * TPW
    pltpu.sync_copy(pos0_hbm.at[pl.ds(base, TPW)], idx0)
    pltpu.sync_copy(pos1_hbm.at[pl.ds(base, TPW)], idx1)
    pltpu.async_copy(yg_hbm.at[idx0], r0, sem).wait()
    pltpu.async_copy(yg_hbm.at[idx1], r1, sem).wait()
    pltpu.sync_copy(r0, g0_hbm.at[pl.ds(base, TPW)])
    pltpu.sync_copy(r1, g1_hbm.at[pl.ds(base, TPW)])


def _sc_gather(yg, pos0, pos1):
    mesh = plsc.VectorSubcoreMesh(
        core_axis_name="c", subcore_axis_name="s", num_cores=NC, num_subcores=NS
    )
    f = functools.partial(
        pl.kernel,
        out_type=[
            jax.ShapeDtypeStruct((N, D), jnp.float32),
            jax.ShapeDtypeStruct((N, D), jnp.float32),
        ],
        mesh=mesh,
        scratch_types=[
            pltpu.VMEM((TPW,), jnp.int32),
            pltpu.VMEM((TPW,), jnp.int32),
            pltpu.VMEM((TPW, D), jnp.float32),
            pltpu.VMEM((TPW, D), jnp.float32),
            pltpu.SemaphoreType.DMA,
        ],
    )(_scb_body)
    return f(yg, pos0, pos1)


def _add_body(a_ref, b_ref, o_ref):
    o_ref[...] = a_ref[...] + b_ref[...]


def kernel(x, g1, bn1, Wqkv, bqkv, Wp, bp, g2, bn2, Wr, W1, b1, W2, b2):
    xf = x.reshape(N, D)
    g1r, bn1r = g1.reshape(1, D), bn1.reshape(1, D)
    g2r, bn2r = g2.reshape(1, D), bn2.reshape(1, D)
    bqkvr = bqkv.reshape(1, 3 * D)
    bpr = bp.reshape(1, D)
    Wr_pad = jnp.pad(Wr, ((0, 0), (0, EP - E)))
    b1r = b1.reshape(E, 2, 1, HID // 2)
    b2r = b2.reshape(E, 1, 1, D)

    qkv = pl.pallas_call(
        _ln_qkv_body,
        grid=(N // RB, 3),
        in_specs=[
            pl.BlockSpec((RB, D), lambda i, c: (i, 0)),
            pl.BlockSpec((1, D), lambda i, c: (0, 0)),
            pl.BlockSpec((1, D), lambda i, c: (0, 0)),
            pl.BlockSpec((D, D), lambda i, c: (0, c)),
            pl.BlockSpec((1, D), lambda i, c: (0, c)),
        ],
        out_specs=pl.BlockSpec((RB, D), lambda i, c: (i, c)),
        out_shape=jax.ShapeDtypeStruct((N, 3 * D), jnp.float32),
    )(xf, g1r, bn1r, Wqkv, bqkvr)

    NP = H // 2
    o3p = pl.pallas_call(
        _attn_body,
        grid=(NP, N // RB),
        in_specs=[
            pl.BlockSpec((RB, 2 * DH), lambda p, i: (i, p)),
            pl.BlockSpec((N, 2 * DH), lambda p, i: (0, NP + p)),
            pl.BlockSpec((N, 2 * DH), lambda p, i: (0, 2 * NP + p)),
        ],
        out_specs=pl.BlockSpec((RB, 2 * DH), lambda p, i: (i, p)),
        out_shape=jax.ShapeDtypeStruct((N, D), jnp.float32),
    )(qkv, qkv, qkv)

    x2, ridx, rwt = pl.pallas_call(
        _post_body,
        grid=(N // RB,),
        in_specs=[
            pl.BlockSpec((RB, D), lambda i: (i, 0)),
            pl.BlockSpec((D, D), lambda i: (0, 0)),
            pl.BlockSpec((1, D), lambda i: (0, 0)),
            pl.BlockSpec((RB, D), lambda i: (i, 0)),
            pl.BlockSpec((1, D), lambda i: (0, 0)),
            pl.BlockSpec((1, D), lambda i: (0, 0)),
            pl.BlockSpec((D, EP), lambda i: (0, 0)),
        ],
        out_specs=[
            pl.BlockSpec((RB, D), lambda i: (i, 0)),
            pl.BlockSpec((RB, EP), lambda i: (i, 0)),
            pl.BlockSpec((RB, EP), lambda i: (i, 0)),
        ],
        out_shape=[
            jax.ShapeDtypeStruct((N, D), jnp.float32),
            jax.ShapeDtypeStruct((N, EP), jnp.int32),
            jax.ShapeDtypeStruct((N, EP), jnp.float32),
        ],
    )(o3p, Wp, bpr, xf, g2r, bn2r, Wr_pad)

    pos, meta8, w0rep, w1rep = pl.pallas_call(
        _meta_body,
        grid=(1,),
        in_specs=[
            pl.BlockSpec((N, EP), lambda i: (0, 0)),
            pl.BlockSpec((N, EP), lambda i: (0, 0)),
        ],
        out_specs=[
            pl.BlockSpec((A, EP), lambda i: (0, 0)),
            pl.BlockSpec((8, EP), lambda i: (0, 0)),
            pl.BlockSpec((N, EP), lambda i: (0, 0)),
            pl.BlockSpec((N, EP), lambda i: (0, 0)),
        ],
        out_shape=[
            jax.ShapeDtypeStruct((A, EP), jnp.int32),
            jax.ShapeDtypeStruct((8, EP), jnp.int32),
            jax.ShapeDtypeStruct((N, EP), jnp.float32),
            jax.ShapeDtypeStruct((N, EP), jnp.float32),
        ],
    )(ridx, rwt)

    pos0 = pos[:N, 0]
    pos1 = pos[N:, 0]
    bmeta = meta8[:2, :NBLK].transpose(1, 0)

    xg, wg = _sc_scatter(x2, pos0, pos1, w0rep, w1rep)

    grid_spec = pltpu.PrefetchScalarGridSpec(
        num_scalar_prefetch=1,
        grid=(NBLK, 2),
        in_specs=[
            pl.BlockSpec((GB, D), lambda b, c, m: (b, 0)),
            pl.BlockSpec((GB, EP), lambda b, c, m: (b, 0)),
            pl.BlockSpec((1, D, HID // 2), lambda b, c, m: (m[b, 0], 0, c)),
            pl.BlockSpec((1, 1, 1, HID // 2), lambda b, c, m: (m[b, 0], c, 0, 0)),
            pl.BlockSpec((1, HID // 2, D), lambda b, c, m: (m[b, 0], c, 0)),
            pl.BlockSpec((1, 1, 1, D), lambda b, c, m: (m[b, 0], 0, 0, 0)),
        ],
        out_specs=pl.BlockSpec((GB, D), lambda b, c, m: (b, 0)),
    )
    yg = pl.pallas_call(
        _ffn_body,
        grid_spec=grid_spec,
        out_shape=jax.ShapeDtypeStruct((P, D), jnp.float32),
    )(bmeta, xg, wg, W1, b1r, W2, b2r)

    g0, g1 = _sc_gather(yg, pos0, pos1)

    out = pl.pallas_call(
        _add_body,
        grid=(N // RB,),
        in_specs=[
            pl.BlockSpec((RB, D), lambda i: (i, 0)),
            pl.BlockSpec((RB, D), lambda i: (i, 0)),
        ],
        out_specs=pl.BlockSpec((RB, D), lambda i: (i, 0)),
        out_shape=jax.ShapeDtypeStruct((N, D), jnp.float32),
    )(g0, g1)

    return out.reshape(1, N, D)
